# baseline (speedup 1.0000x reference)
import numpy as np
import jax
import jax.numpy as jnp
from jax.experimental import pallas as pl

_NR = 5000
_NL = 20000
_NLP = 20096
_K = 8
_T = 10
_R = 40
_LOG2PI = float(np.log(2.0 * np.pi))
_W_OCC = 0.2
_W_MDN = 1.0
_W_INT = 0.1
_PAD_S = 65535.0


def _body(raug_ref, lmat_ref, gmat_ref, par_ref, out_ref):
    i = pl.program_id(0)

    @pl.when(i == 0)
    def _init():
        out_ref[...] = jnp.zeros((1, 1), jnp.float32)

    lm = lmat_ref[...]
    l0 = lm[0:1, :]
    l1 = lm[1:2, :]
    l2c = lm[2:3, :]
    lpad = lm[4:5, :]
    lsq = l0 * l0 + l1 * l1 + l2c * l2c + lpad

    rdl = jnp.dot(raug_ref[...], lm, preferred_element_type=jnp.float32)
    s = lsq + rdl
    packed = s.astype(jnp.int32) * 32768 + jax.lax.broadcasted_iota(
        jnp.int32, s.shape, 1)

    par = par_ref[...]
    mu = (par[:, 0:8], par[:, 8:16], par[:, 16:24])
    ls = (par[:, 24:32], par[:, 32:40], par[:, 40:48])
    mix = par[:, 48:56]
    occ = par[:, 56:64]
    mui = par[:, 64:72]
    rp = (par[:, 72:73], par[:, 73:74], par[:, 74:75])

    inv_s2 = tuple(1.0 / (jnp.exp(2.0 * l) + 1e-12) for l in ls)
    mmax = jnp.max(mix, axis=1, keepdims=True)
    lpi = mix - mmax - jnp.log(jnp.sum(jnp.exp(mix - mmax), axis=1,
                                       keepdims=True))

    gm = gmat_ref[...]
    mdn_part = jnp.float32(0.0)
    int_part = jnp.float32(0.0)
    for _ in range(_T):
        m = jnp.min(packed, axis=1, keepdims=True)
        mask = packed == m
        g = jnp.dot(mask.astype(jnp.float32), gm,
                    preferred_element_type=jnp.float32,
                    precision=jax.lax.Precision.HIGHEST)
        packed = jnp.where(mask, jnp.int32(2147483647), packed)

        gt_int = (g[:, 3:4] + g[:, 7:8] + g[:, 11:12] + g[:, 15:16]) * 0.25
        y = (g[:, 18:19] - rp[2], g[:, 17:18] - rp[1], g[:, 16:17] - rp[0])
        quad = sum(((y[d] - mu[d]) ** 2) * inv_s2[d] + 2.0 * ls[d]
                   for d in range(3))
        logn = -0.5 * (quad + 3.0 * _LOG2PI)
        lmix = logn + lpi
        mx = jnp.max(lmix, axis=1, keepdims=True)
        e = jnp.exp(lmix - mx)
        se = jnp.sum(e, axis=1, keepdims=True)
        mdn_part += jnp.sum(mx + jnp.log(se))
        int_part += jnp.sum((e / se) * jnp.abs(mui - gt_int))

    occ_any = jnp.max(occ, axis=1)
    z = -occ_any
    occ_part = jnp.sum(jnp.maximum(z, 0.0) + jnp.log(1.0 + jnp.exp(-jnp.abs(z))))

    total = ((_W_OCC / _NR) * occ_part
             + (-_W_MDN / (_NR * _T)) * mdn_part
             + (_W_INT / (_NR * _T * _K)) * int_part)
    out_ref[...] += jnp.reshape(total, (1, 1))


def _fused(raug, lmat, gmat, par):
    return pl.pallas_call(
        _body,
        grid=(_NR // _R,),
        in_specs=[
            pl.BlockSpec((_R, 8), lambda i: (i, 0)),
            pl.BlockSpec((8, _NLP), lambda i: (0, 0)),
            pl.BlockSpec((_NLP, 24), lambda i: (0, 0)),
            pl.BlockSpec((_R, 80), lambda i: (i, 0)),
        ],
        out_specs=pl.BlockSpec((1, 1), lambda i: (0, 0)),
        out_shape=jax.ShapeDtypeStruct((1, 1), jnp.float32),
    )(raug, lmat, gmat, par)


def kernel(mu_off, log_sig_off, mu_int, occ_logit, mix_logit,
           radar_indices, radar_features, lidar_indices, lidar_features):
    rpos = radar_indices[:, 1:].astype(jnp.float32)
    lpos = lidar_indices[:, 1:].astype(jnp.float32)
    padc = _NLP - _NL

    raug = jnp.concatenate(
        [-2.0 * rpos, jnp.zeros((_NR, 5), jnp.float32)], axis=1)

    lmat = jnp.concatenate([
        jnp.pad(lpos.T, ((0, 0), (0, padc))),
        jnp.zeros((1, _NLP), jnp.float32),
        jnp.pad(jnp.zeros((1, _NL), jnp.float32), ((0, 0), (0, padc)),
                constant_values=_PAD_S),
        jnp.zeros((3, _NLP), jnp.float32),
    ], axis=0)

    gmat = jnp.concatenate([
        jnp.pad(lidar_features, ((0, padc), (0, 0))),
        jnp.pad(lpos, ((0, padc), (0, 0))),
        jnp.zeros((_NLP, 5), jnp.float32),
    ], axis=1)

    mu_t = mu_off.transpose(0, 2, 1).reshape(_NR, 24)
    ls_t = log_sig_off.transpose(0, 2, 1).reshape(_NR, 24)
    par = jnp.concatenate([
        mu_t, ls_t,
        mix_logit[..., 0], occ_logit[..., 0], mu_int[..., 0],
        rpos, jnp.zeros((_NR, 5), jnp.float32),
    ], axis=1)

    out = _fused(raug, lmat, gmat, par)
    return out[0, 0]


# speedup vs baseline: 4.9816x; 4.9816x over previous
import numpy as np
import jax
import jax.numpy as jnp
from jax.experimental import pallas as pl

_NR = 5000
_NL = 20000
_NLP = 20096
_K = 8
_T = 10
_R = 40
_LOG2PI = float(np.log(2.0 * np.pi))
_W_OCC = 0.2
_W_MDN = 1.0
_W_INT = 0.1
_PAD_S = 65535.0


def _body(raug_ref, lmat_ref, gmat_ref, par_ref, out_ref):
    i = pl.program_id(0)

    @pl.when(i == 0)
    def _init():
        out_ref[...] = jnp.zeros((1, 1), jnp.float32)

    lm = lmat_ref[...]
    l0 = lm[0:1, :]
    l1 = lm[1:2, :]
    l2c = lm[2:3, :]
    lpad = lm[4:5, :]
    lsq = l0 * l0 + l1 * l1 + l2c * l2c + lpad

    rdl = jnp.dot(raug_ref[...], lm, preferred_element_type=jnp.float32)
    s = lsq + rdl
    packed = s.astype(jnp.int32) * 32768 + jax.lax.broadcasted_iota(
        jnp.int32, s.shape, 1)

    par = par_ref[...]
    mu = (par[:, 0:8], par[:, 8:16], par[:, 16:24])
    ls = (par[:, 24:32], par[:, 32:40], par[:, 40:48])
    mix = par[:, 48:56]
    occ = par[:, 56:64]
    mui = par[:, 64:72]
    rp = (par[:, 72:73], par[:, 73:74], par[:, 74:75])

    inv_s2 = tuple(1.0 / (jnp.exp(2.0 * l) + 1e-12) for l in ls)
    mmax = jnp.max(mix, axis=1, keepdims=True)
    lpi = mix - mmax - jnp.log(jnp.sum(jnp.exp(mix - mmax), axis=1,
                                       keepdims=True))

    gm = gmat_ref[...]
    mdn_part = jnp.float32(0.0)
    int_part = jnp.float32(0.0)
    for _ in range(_T):
        m = jnp.min(packed, axis=1, keepdims=True)
        mask = packed == m
        g = jnp.dot(mask.astype(jnp.float32), gm,
                    preferred_element_type=jnp.float32)
        packed = jnp.where(mask, jnp.int32(2147483647), packed)

        gt_int = (g[:, 3:4] + g[:, 7:8] + g[:, 11:12] + g[:, 15:16]) * 0.25
        y = (g[:, 18:19] - rp[2], g[:, 17:18] - rp[1], g[:, 16:17] - rp[0])
        quad = sum(((y[d] - mu[d]) ** 2) * inv_s2[d] + 2.0 * ls[d]
                   for d in range(3))
        logn = -0.5 * (quad + 3.0 * _LOG2PI)
        lmix = logn + lpi
        mx = jnp.max(lmix, axis=1, keepdims=True)
        e = jnp.exp(lmix - mx)
        se = jnp.sum(e, axis=1, keepdims=True)
        mdn_part += jnp.sum(mx + jnp.log(se))
        int_part += jnp.sum((e / se) * jnp.abs(mui - gt_int))

    occ_any = jnp.max(occ, axis=1)
    z = -occ_any
    occ_part = jnp.sum(jnp.maximum(z, 0.0) + jnp.log(1.0 + jnp.exp(-jnp.abs(z))))

    total = ((_W_OCC / _NR) * occ_part
             + (-_W_MDN / (_NR * _T)) * mdn_part
             + (_W_INT / (_NR * _T * _K)) * int_part)
    out_ref[...] += jnp.reshape(total, (1, 1))


def _fused(raug, lmat, gmat, par):
    return pl.pallas_call(
        _body,
        grid=(_NR // _R,),
        in_specs=[
            pl.BlockSpec((_R, 8), lambda i: (i, 0)),
            pl.BlockSpec((8, _NLP), lambda i: (0, 0)),
            pl.BlockSpec((_NLP, 24), lambda i: (0, 0)),
            pl.BlockSpec((_R, 80), lambda i: (i, 0)),
        ],
        out_specs=pl.BlockSpec((1, 1), lambda i: (0, 0)),
        out_shape=jax.ShapeDtypeStruct((1, 1), jnp.float32),
    )(raug, lmat, gmat, par)


def kernel(mu_off, log_sig_off, mu_int, occ_logit, mix_logit,
           radar_indices, radar_features, lidar_indices, lidar_features):
    rpos = radar_indices[:, 1:].astype(jnp.float32)
    lpos = lidar_indices[:, 1:].astype(jnp.float32)
    padc = _NLP - _NL

    raug = jnp.concatenate(
        [-2.0 * rpos, jnp.zeros((_NR, 5), jnp.float32)], axis=1)

    lmat = jnp.concatenate([
        jnp.pad(lpos.T, ((0, 0), (0, padc))),
        jnp.zeros((1, _NLP), jnp.float32),
        jnp.pad(jnp.zeros((1, _NL), jnp.float32), ((0, 0), (0, padc)),
                constant_values=_PAD_S),
        jnp.zeros((3, _NLP), jnp.float32),
    ], axis=0)

    gmat = jnp.concatenate([
        jnp.pad(lidar_features, ((0, padc), (0, 0))),
        jnp.pad(lpos, ((0, padc), (0, 0))),
        jnp.zeros((_NLP, 5), jnp.float32),
    ], axis=1)

    mu_t = mu_off.transpose(0, 2, 1).reshape(_NR, 24)
    ls_t = log_sig_off.transpose(0, 2, 1).reshape(_NR, 24)
    par = jnp.concatenate([
        mu_t, ls_t,
        mix_logit[..., 0], occ_logit[..., 0], mu_int[..., 0],
        rpos, jnp.zeros((_NR, 5), jnp.float32),
    ], axis=1)

    out = _fused(raug, lmat, gmat, par)
    return out[0, 0]
